# baseline (speedup 1.0000x reference)
import functools
import math

import jax
import jax.numpy as jnp
from jax import lax
from jax.experimental import pallas as pl
from jax.experimental.pallas import tpu as pltpu
from jax.experimental.pallas import tpu_sc as plsc

_VOCAB = 1000000
_D = 64
_B = 4096
_H = 200
_NW = 32
_RPW = _B // _NW
_SZ = (104, 96)
_OFF = (0, 104)
_NSUB = _RPW * 2
_SCALE = math.sqrt(_D)
_NBUF = 8
_LOOK = 3


def kernel(x, lut):
    mesh = plsc.VectorSubcoreMesh(core_axis_name="c", subcore_axis_name="s")

    @functools.partial(
        pl.kernel,
        mesh=mesh,
        out_type=jax.ShapeDtypeStruct((_B, _H, _D), jnp.float32),
        scratch_types=[
            pltpu.VMEM((_RPW, _H), jnp.int32),
            [pltpu.VMEM((_SZ[s % 2], _D), jnp.float32) for s in range(_NBUF)],
            [pltpu.SemaphoreType.DMA for _ in range(_NBUF)],
            [pltpu.SemaphoreType.DMA for _ in range(_NBUF)],
        ],
        compiler_params=pltpu.CompilerParams(use_tc_tiling_on_sc=False),
    )
    def k(x_hbm, lut_hbm, out_hbm, idx_v, bufs, gsems, ssems):
        wid = lax.axis_index("s") * 2 + lax.axis_index("c")
        row0 = wid * _RPW
        pltpu.sync_copy(x_hbm.at[pl.ds(row0, _RPW), :], idx_v)

        def gather_src(r, h):
            return lut_hbm.at[idx_v.at[r, pl.ds(_OFF[h], _SZ[h])]]

        def out_dst(r, h):
            return out_hbm.at[row0 + r, pl.ds(_OFF[h], _SZ[h]), :]

        for c in range(_LOOK):
            pltpu.async_copy(gather_src(c // 2, c % 2), bufs[c], gsems[c])

        def tick(i, carry):
            c0 = i * _NBUF
            for s in range(_NBUF):
                c = c0 + s
                h = s % 2
                sg = (s + _LOOK) % _NBUF
                hg = sg % 2
                cg = c + _LOOK

                @pl.when(cg >= _NBUF)
                def _():
                    pltpu.make_async_copy(
                        bufs[sg], out_dst((cg - _NBUF) // 2, hg),
                        ssems[sg]).wait()

                @pl.when(cg < _NSUB)
                def _():
                    pltpu.async_copy(gather_src(cg // 2, hg), bufs[sg],
                                     gsems[sg])

                pltpu.make_async_copy(gather_src(c // 2, h), bufs[s],
                                      gsems[s]).wait()

                def srow(r, c2):
                    for col in range(_D // 16):
                        sl = pl.ds(col * 16, 16)
                        bufs[s][r, sl] = bufs[s][r, sl] * _SCALE
                    return c2

                lax.fori_loop(0, _SZ[h], srow, 0, unroll=4)
                pltpu.async_copy(bufs[s], out_dst(c // 2, h), ssems[s])
            return carry

        lax.fori_loop(0, _NSUB // _NBUF, tick, 0)

        for c in range(_NSUB - (_NBUF - _LOOK), _NSUB):
            s = c % _NBUF
            pltpu.make_async_copy(bufs[s], out_dst(c // 2, s % 2),
                                  ssems[s]).wait()

    return k(x, lut)
